# baseline (speedup 1.0000x reference)
import functools

import jax
import jax.numpy as jnp
from jax import lax
from jax.experimental import pallas as pl
from jax.experimental.pallas import tpu as pltpu
from jax.experimental.pallas import tpu_sc as plsc

Q = 1024
D = 64
N = 100000
KB = 2000
NB = N // KB


def _normalize(x, eps=1e-12):
    n = jnp.sqrt(jnp.sum(x * x, axis=-1, keepdims=True))
    return x / jnp.maximum(n, eps)


SQ = 256
KBP = 2048
T = KBP // 128


def _stats_body(q_ref, k_ref, idx_out, var_out,
                qn_s, max_s, gid_s, fst_s, ksum_s, gram_s):
    i = pl.program_id(0)

    @pl.when(i == 0)
    def _init():
        qn_s[...] = _normalize(q_ref[...])
        max_s[...] = jnp.full((Q, 128), -jnp.inf, jnp.float32)
        gid_s[...] = jnp.zeros((Q, 128), jnp.int32)
        fst_s[...] = jnp.full((Q, 128), -jnp.inf, jnp.float32)
        ksum_s[...] = jnp.zeros((1, D), jnp.float32)
        gram_s[...] = jnp.zeros((D, D), jnp.float32)

    kn = _normalize(k_ref[...])
    knp = jnp.concatenate(
        [kn, jnp.broadcast_to(kn[0:1, :], (KBP - KB, D))], axis=0)
    sim = lax.dot_general(qn_s[...], knp, (((1,), (1,)), ((), ())),
                          preferred_element_type=jnp.float32)

    for s in range(Q // SQ):
        rows = pl.ds(s * SQ, SQ)
        m = max_s[rows, :]
        g = gid_s[rows, :]
        f = fst_s[rows, :]
        for p in range(T // 2):
            st0 = sim[s * SQ:(s + 1) * SQ, (2 * p) * 128:(2 * p + 1) * 128]
            st1 = sim[s * SQ:(s + 1) * SQ, (2 * p + 1) * 128:(2 * p + 2) * 128]
            m2 = jnp.maximum(st0, st1)
            upd = m2 > m
            m = jnp.maximum(m2, m)
            g = jnp.where(upd, jnp.full_like(g, i * (T // 2) + p), g)
            f = jnp.where(upd, st0, f)
        max_s[rows, :] = m
        gid_s[rows, :] = g
        fst_s[rows, :] = f

    ksum_s[...] += jnp.sum(kn, axis=0, keepdims=True)
    gram_s[...] += lax.dot_general(kn, kn, (((0,), (0,)), ((), ())),
                                   preferred_element_type=jnp.float32)

    @pl.when(i == NB - 1)
    def _fin():
        m = max_s[...]
        g = gid_s[...]
        second = (fst_s[...] != m).astype(jnp.int32)
        lane = lax.broadcasted_iota(jnp.int32, (Q, 128), 1)
        col = ((g // (T // 2)) * KB + ((g % (T // 2)) * 2 + second) * 128
               + lane)
        gmax = jnp.max(m, axis=1, keepdims=True)
        idx_out[...] = jnp.min(
            jnp.where(m == gmax, col, jnp.int32(2**30)),
            axis=1, keepdims=True)
        qn = qn_s[...]
        s = lax.dot_general(qn, ksum_s[...], (((1,), (1,)), ((), ())),
                            preferred_element_type=jnp.float32)
        qg = lax.dot_general(qn, gram_s[...], (((1,), (0,)), ((), ())),
                             preferred_element_type=jnp.float32)
        ss = jnp.sum(qg * qn, axis=1, keepdims=True)
        var_rows = (ss - s * s / N) / (N - 1)
        var_out[...] = jnp.full((1, 1), jnp.mean(var_rows), jnp.float32)


def _topk_stats(query, keys, interpret=False):
    return pl.pallas_call(
        _stats_body,
        grid=(NB,),
        in_specs=[
            pl.BlockSpec((Q, D), lambda i: (0, 0)),
            pl.BlockSpec((KB, D), lambda i: (i, 0)),
        ],
        out_specs=[
            pl.BlockSpec((Q, 1), lambda i: (0, 0)),
            pl.BlockSpec((1, 1), lambda i: (0, 0)),
        ],
        out_shape=[
            jax.ShapeDtypeStruct((Q, 1), jnp.int32),
            jax.ShapeDtypeStruct((1, 1), jnp.float32),
        ],
        scratch_shapes=[
            pltpu.VMEM((Q, D), jnp.float32),
            pltpu.VMEM((Q, 128), jnp.float32),
            pltpu.VMEM((Q, 128), jnp.int32),
            pltpu.VMEM((Q, 128), jnp.float32),
            pltpu.VMEM((1, D), jnp.float32),
            pltpu.VMEM((D, D), jnp.float32),
        ],
        interpret=interpret,
    )(query, keys)


def _sc_gather(values, idx):
    info = plsc.get_sparse_core_info()
    nw = info.num_cores * info.num_subcores
    b_per_w = Q // nw
    mesh = plsc.VectorSubcoreMesh(core_axis_name="c", subcore_axis_name="s")

    @functools.partial(
        pl.kernel, mesh=mesh,
        out_type=jax.ShapeDtypeStruct((Q, D), jnp.float32),
        scratch_types=[
            pltpu.VMEM((b_per_w,), jnp.int32),
            pltpu.VMEM((b_per_w, D), jnp.float32),
            pltpu.SemaphoreType.DMA,
        ],
        compiler_params=pltpu.CompilerParams(use_tc_tiling_on_sc=False),
    )
    def gather_k(table_hbm, idx_hbm, out_hbm, idx_v, rows_v, sem):
        wid = lax.axis_index("s") * info.num_cores + lax.axis_index("c")
        base = wid * b_per_w
        pltpu.sync_copy(idx_hbm.at[pl.ds(base, b_per_w)], idx_v)
        pltpu.async_copy(table_hbm.at[idx_v], rows_v, sem).wait()
        pltpu.sync_copy(rows_v, out_hbm.at[pl.ds(base, b_per_w)])

    return gather_k(values, idx)


def kernel(query, keys, values):
    query = query.astype(jnp.float32)
    keys = keys.astype(jnp.float32)
    values = values.astype(jnp.float32)
    idx, var = _topk_stats(query, keys)
    retrieved = _sc_gather(values, idx.reshape(Q))
    return (retrieved, var.reshape(()))


# speedup vs baseline: 1.0772x; 1.0772x over previous
import functools

import jax
import jax.numpy as jnp
from jax import lax
from jax.experimental import pallas as pl
from jax.experimental.pallas import tpu as pltpu
from jax.experimental.pallas import tpu_sc as plsc

Q = 1024
D = 64
N = 100000
KB = 4000
NB = N // KB


def _normalize(x, eps=1e-12):
    n = jnp.sqrt(jnp.sum(x * x, axis=-1, keepdims=True))
    return x / jnp.maximum(n, eps)


SQ = 256
KBP = 4096
T = KBP // 128


def _stats_body(q_ref, k_ref, vlo_ref, vhi_ref, idx_out, var_out, vpack_out,
                qn_s, max_s, gid_s, fst_s, ksum_s, gram_s):
    i = pl.program_id(0)

    @pl.when(i == 0)
    def _init():
        qn_s[...] = _normalize(q_ref[...])
        max_s[...] = jnp.full((Q, 128), -jnp.inf, jnp.float32)
        gid_s[...] = jnp.zeros((Q, 128), jnp.int32)
        fst_s[...] = jnp.full((Q, 128), -jnp.inf, jnp.float32)
        ksum_s[...] = jnp.zeros((1, D), jnp.float32)
        gram_s[...] = jnp.zeros((D, D), jnp.float32)

    vpack_out[...] = jnp.concatenate([vlo_ref[...], vhi_ref[...]], axis=1)

    kn = _normalize(k_ref[...])
    knp = jnp.concatenate(
        [kn, jnp.broadcast_to(kn[0:1, :], (KBP - KB, D))], axis=0)
    sim = lax.dot_general(qn_s[...], knp, (((1,), (1,)), ((), ())),
                          preferred_element_type=jnp.float32)

    for s in range(Q // SQ):
        rows = pl.ds(s * SQ, SQ)
        m = max_s[rows, :]
        g = gid_s[rows, :]
        f = fst_s[rows, :]
        for p in range(T // 2):
            st0 = sim[s * SQ:(s + 1) * SQ, (2 * p) * 128:(2 * p + 1) * 128]
            st1 = sim[s * SQ:(s + 1) * SQ, (2 * p + 1) * 128:(2 * p + 2) * 128]
            m2 = jnp.maximum(st0, st1)
            upd = m2 > m
            m = jnp.maximum(m2, m)
            g = jnp.where(upd, jnp.full_like(g, i * (T // 2) + p), g)
            f = jnp.where(upd, st0, f)
        max_s[rows, :] = m
        gid_s[rows, :] = g
        fst_s[rows, :] = f

    ksum_s[...] += jnp.sum(kn, axis=0, keepdims=True)
    gram_s[...] += lax.dot_general(kn, kn, (((0,), (0,)), ((), ())),
                                   preferred_element_type=jnp.float32)

    @pl.when(i == NB - 1)
    def _fin():
        m = max_s[...]
        g = gid_s[...]
        second = (fst_s[...] != m).astype(jnp.int32)
        lane = lax.broadcasted_iota(jnp.int32, (Q, 128), 1)
        col = ((g // (T // 2)) * KB + ((g % (T // 2)) * 2 + second) * 128
               + lane)
        gmax = jnp.max(m, axis=1, keepdims=True)
        idx_out[...] = jnp.min(
            jnp.where(m == gmax, col, jnp.int32(2**30)),
            axis=1, keepdims=True)
        qn = qn_s[...]
        s = lax.dot_general(qn, ksum_s[...], (((1,), (1,)), ((), ())),
                            preferred_element_type=jnp.float32)
        qg = lax.dot_general(qn, gram_s[...], (((1,), (0,)), ((), ())),
                             preferred_element_type=jnp.float32)
        ss = jnp.sum(qg * qn, axis=1, keepdims=True)
        var_rows = (ss - s * s / N) / (N - 1)
        var_out[...] = jnp.full((1, 1), jnp.mean(var_rows), jnp.float32)


def _topk_stats(query, keys, values, interpret=False):
    return pl.pallas_call(
        _stats_body,
        grid=(NB,),
        in_specs=[
            pl.BlockSpec((Q, D), lambda i: (0, 0)),
            pl.BlockSpec((KB, D), lambda i: (i, 0)),
            pl.BlockSpec((KB // 2, D), lambda i: (i, 0)),
            pl.BlockSpec((KB // 2, D), lambda i: (i + NB, 0)),
        ],
        out_specs=[
            pl.BlockSpec((Q, 1), lambda i: (0, 0)),
            pl.BlockSpec((1, 1), lambda i: (0, 0)),
            pl.BlockSpec((KB // 2, 2 * D), lambda i: (i, 0)),
        ],
        out_shape=[
            jax.ShapeDtypeStruct((Q, 1), jnp.int32),
            jax.ShapeDtypeStruct((1, 1), jnp.float32),
            jax.ShapeDtypeStruct((N // 2, 2 * D), jnp.float32),
        ],
        scratch_shapes=[
            pltpu.VMEM((Q, D), jnp.float32),
            pltpu.VMEM((Q, 128), jnp.float32),
            pltpu.VMEM((Q, 128), jnp.int32),
            pltpu.VMEM((Q, 128), jnp.float32),
            pltpu.VMEM((1, D), jnp.float32),
            pltpu.VMEM((D, D), jnp.float32),
        ],
        interpret=interpret,
    )(query, keys, values, values)


def _sc_gather_pairs(values2, idx):
    info = plsc.get_sparse_core_info()
    nw = info.num_cores * info.num_subcores
    b_per_w = Q // nw
    mesh = plsc.VectorSubcoreMesh(core_axis_name="c", subcore_axis_name="s")

    @functools.partial(
        pl.kernel, mesh=mesh,
        out_type=jax.ShapeDtypeStruct((Q, 2 * D), jnp.float32),
        scratch_types=[
            pltpu.VMEM((b_per_w,), jnp.int32),
            pltpu.VMEM((b_per_w,), jnp.int32),
            pltpu.VMEM((b_per_w, 2 * D), jnp.float32),
            pltpu.SemaphoreType.DMA,
        ],
    )
    def gather_k(table_hbm, idx_hbm, out_hbm, idx_v, pair_v, rows_v, sem):
        wid = lax.axis_index("s") * info.num_cores + lax.axis_index("c")
        base = wid * b_per_w
        pltpu.sync_copy(idx_hbm.at[pl.ds(base, b_per_w)], idx_v)
        for c in range(b_per_w // 16):
            sl = pl.ds(c * 16, 16)
            ix = idx_v[sl]
            pair_v[sl] = jnp.where(ix >= N // 2, ix - N // 2, ix)
        pltpu.async_copy(table_hbm.at[pair_v], rows_v, sem).wait()
        pltpu.sync_copy(rows_v, out_hbm.at[pl.ds(base, b_per_w)])

    return gather_k(values2, idx)


def _half_select_body(rows_ref, idx_ref, out_ref):
    odd = idx_ref[...] >= N // 2
    lo = rows_ref[:, :D]
    hi = rows_ref[:, D:]
    out_ref[...] = jnp.where(odd, hi, lo)


def _half_select(rows, idx):
    return pl.pallas_call(
        _half_select_body,
        out_shape=jax.ShapeDtypeStruct((Q, D), jnp.float32),
    )(rows, idx)


def kernel(query, keys, values):
    query = query.astype(jnp.float32)
    keys = keys.astype(jnp.float32)
    values = values.astype(jnp.float32)
    idx, var, vpack = _topk_stats(query, keys, values)
    rows = _sc_gather_pairs(vpack, idx.reshape(Q))
    retrieved = _half_select(rows, idx)
    return (retrieved, var.reshape(()))
